# baseline (speedup 1.0000x reference)
import functools

import jax
import jax.numpy as jnp
from jax import lax
from jax.experimental import pallas as pl
from jax.experimental.pallas import tpu as pltpu
from jax.experimental.pallas import tpu_sc as plsc

DIM = 64
PAIR = 2 * DIM
ROWS = 100001
PROWS = (ROWS + 1) // 2
BATCH = 16384
NC = 2
NS = 16
NW = NC * NS
B_PER_W = BATCH // NW
CHUNK = 128
N_CHUNKS = B_PER_W // CHUNK
GROUPS_PER_CHUNK = CHUNK // 16


def _make_gather():
    mesh = plsc.VectorSubcoreMesh(core_axis_name="c", subcore_axis_name="s")

    @functools.partial(
        pl.kernel,
        mesh=mesh,
        out_type=jax.ShapeDtypeStruct((BATCH // 2, PAIR), jnp.float32),
        scratch_types=[
            pltpu.VMEM((B_PER_W,), jnp.int32),
            pltpu.VMEM((B_PER_W,), jnp.int32),
            pltpu.VMEM((B_PER_W, PAIR), jnp.float32),
            pltpu.VMEM((B_PER_W // 2, PAIR), jnp.float32),
            pltpu.SemaphoreType.DMA,
        ],
        compiler_params=pltpu.CompilerParams(
            use_tc_tiling_on_sc=True, needs_layout_passes=False
        ),
    )
    def gather_kernel(table_hbm, idx_hbm, out_hbm, idx_v, pidx_v, rows_v,
                      pack_v, sem):
        wid = lax.axis_index("s") * NC + lax.axis_index("c")
        base = wid * B_PER_W
        pltpu.sync_copy(idx_hbm.at[pl.ds(base, B_PER_W)], idx_v)
        lane = lax.iota(jnp.int32, 16)

        def shift_body(k, carry):
            v = idx_v[pl.ds(k * 16, 16)]
            pidx_v[pl.ds(k * 16, 16)] = v >> 1
            return carry

        lax.fori_loop(0, B_PER_W // 16, shift_body, 0)

        gathers = [
            pltpu.async_copy(
                table_hbm.at[pidx_v.at[pl.ds(j * CHUNK, CHUNK)]],
                rows_v.at[pl.ds(j * CHUNK, CHUNK)],
                sem,
            )
            for j in range(N_CHUNKS)
        ]

        for j in range(N_CHUNKS):
            gathers[j].wait()

            def pack_body(g, carry, _j=j):
                r16 = (_j * GROUPS_PER_CHUNK + g) * 16 + lane
                v16 = idx_v[pl.ds((_j * GROUPS_PER_CHUNK + g) * 16, 16)]
                src_base = (v16 & 1) * DIM
                dst_r = r16 >> 1
                dst_base = (r16 & 1) * DIM
                for c in range(DIM):
                    val = plsc.load_gather(rows_v, [r16, src_base + c])
                    plsc.store_scatter(pack_v, [dst_r, dst_base + c], val)
                return carry

            lax.fori_loop(0, GROUPS_PER_CHUNK, pack_body, 0)

        pltpu.sync_copy(
            pack_v, out_hbm.at[pl.ds(wid * (B_PER_W // 2), B_PER_W // 2)]
        )

    return gather_kernel


_gather = _make_gather()


def kernel(t, emb):
    table = jnp.pad(emb, ((0, 1), (0, 0))).reshape(PROWS, PAIR)
    return _gather(table, t).reshape(BATCH, DIM)


# speedup vs baseline: 1.9632x; 1.9632x over previous
import functools

import jax
import jax.numpy as jnp
from jax import lax
from jax.experimental import pallas as pl
from jax.experimental.pallas import tpu as pltpu
from jax.experimental.pallas import tpu_sc as plsc

DIM = 64
PADDED = 128
BATCH = 16384
NC = 2
NS = 16
NW = NC * NS
B_PER_W = BATCH // NW
CHUNK = 128
N_CHUNKS = B_PER_W // CHUNK


def _make_gather():
    mesh = plsc.VectorSubcoreMesh(core_axis_name="c", subcore_axis_name="s")

    @functools.partial(
        pl.kernel,
        mesh=mesh,
        out_type=jax.ShapeDtypeStruct((BATCH, PADDED), jnp.float32),
        scratch_types=[
            pltpu.VMEM((B_PER_W,), jnp.int32),
            pltpu.VMEM((B_PER_W, PADDED), jnp.float32),
            pltpu.SemaphoreType.DMA,
            pltpu.SemaphoreType.DMA,
        ],
        compiler_params=pltpu.CompilerParams(use_tc_tiling_on_sc=True),
    )
    def gather_kernel(table_hbm, idx_hbm, out_hbm, idx_v, rows_v, g_sem, o_sem):
        wid = lax.axis_index("s") * NC + lax.axis_index("c")
        base = wid * B_PER_W
        pltpu.sync_copy(idx_hbm.at[pl.ds(base, B_PER_W)], idx_v)
        gathers = [
            pltpu.async_copy(
                table_hbm.at[idx_v.at[pl.ds(j * CHUNK, CHUNK)]],
                rows_v.at[pl.ds(j * CHUNK, CHUNK)],
                g_sem,
            )
            for j in range(N_CHUNKS)
        ]
        outs = []
        for j in range(N_CHUNKS):
            gathers[j].wait()
            outs.append(
                pltpu.async_copy(
                    rows_v.at[pl.ds(j * CHUNK, CHUNK)],
                    out_hbm.at[pl.ds(base + j * CHUNK, CHUNK)],
                    o_sem,
                )
            )
        for o in outs:
            o.wait()

    return gather_kernel


_gather = _make_gather()


def kernel(t, emb):
    table = jnp.pad(emb, ((0, 0), (0, PADDED - DIM)))
    return _gather(table, t)[:, :DIM]
